# Initial kernel scaffold; baseline (speedup 1.0000x reference)
import jax
import jax.numpy as jnp
from jax.experimental import pallas as pl


def kernel(L0, L1, L2, X0, X1, X2, batch0, batch1, batch2, W01, b01, W02, b02, W03, b03, W11, b11, W12, b12, W13, b13, W21, b21, W22, b22, W23, b23, Wl, bl):
    raise NotImplementedError("write your pallas kernel here")


# speedup vs baseline: 19.0812x; 19.0812x over previous
import jax
import jax.numpy as jnp
from jax.experimental import pallas as pl
from jax.experimental.pallas import tpu as pltpu

_B = 32


def _dot(a, b_t):
    return jax.lax.dot_general(
        a, b_t, (((1,), (1,)), ((), ())), preferred_element_type=jnp.float32
    )


def _lrelu(t):
    return jnp.where(t > 0, t, 0.01 * t)


def _branch(x, batch2d, w1, b1, w2, b2, w3, b3):
    h = _lrelu(_dot(x, w1) + b1)
    h = _lrelu(_dot(h, w2) + b2)
    h = _dot(h, w3) + b3
    n = x.shape[0]
    seg = jax.lax.broadcasted_iota(jnp.int32, (_B, n), 0)
    mask = (seg == batch2d).astype(jnp.float32)
    sums = jax.lax.dot_general(
        mask, h, (((1,), (0,)), ((), ())), preferred_element_type=jnp.float32
    )
    cnt = jnp.sum(mask, axis=1, keepdims=True)
    return sums / jnp.maximum(cnt, 1.0)


def _body(x0, x1, x2, bt0, bt1, bt2,
          w01, b01, w02, b02, w03, b03,
          w11, b11, w12, b12, w13, b13,
          w21, b21, w22, b22, w23, b23,
          wl, bl, out):
    p0 = _branch(x0[...], bt0[...], w01[...], b01[...], w02[...], b02[...],
                 w03[...], b03[...])
    p1 = _branch(x1[...], bt1[...], w11[...], b11[...], w12[...], b12[...],
                 w13[...], b13[...])
    p2 = _branch(x2[...], bt2[...], w21[...], b21[...], w22[...], b22[...],
                 w23[...], b23[...])
    cat = jnp.concatenate([p0, p1, p2], axis=1)
    logits = _dot(cat, wl[...]) + bl[...]
    m = jnp.max(logits, axis=1, keepdims=True)
    e = jnp.exp(logits - m)
    out[...] = e / jnp.sum(e, axis=1, keepdims=True)


def kernel(L0, L1, L2, X0, X1, X2, batch0, batch1, batch2,
           W01, b01, W02, b02, W03, b03,
           W11, b11, W12, b12, W13, b13,
           W21, b21, W22, b22, W23, b23,
           Wl, bl):
    del L0, L1, L2
    r = lambda v: v.reshape(1, -1)
    out = pl.pallas_call(
        _body,
        out_shape=jax.ShapeDtypeStruct((_B, Wl.shape[0]), jnp.float32),
    )(X0, X1, X2,
      r(batch0), r(batch1), r(batch2),
      W01, r(b01), W02, r(b02), W03, r(b03),
      W11, r(b11), W12, r(b12), W13, r(b13),
      W21, r(b21), W22, r(b22), W23, r(b23),
      Wl, r(bl))
    return out
